# Initial kernel scaffold; baseline (speedup 1.0000x reference)
import jax
import jax.numpy as jnp
from jax.experimental import pallas as pl


def kernel(inputs, Wg, bg, W1, b1, W2, b2):
    raise NotImplementedError("write your pallas kernel here")


# speedup vs baseline: 1.0870x; 1.0870x over previous
import functools

import jax
import jax.numpy as jnp
from jax.experimental import pallas as pl
from jax.experimental.pallas import tpu as pltpu


def _moe_body(x_ref, Wg_ref, bg_ref, W1_ref, b1_ref, W2_ref, b2_ref,
              out_ref, wfull_ref, *, n_exp):
    e = pl.program_id(1)

    @pl.when(e == 0)
    def _gate():
        x = x_ref[...]
        logits = jnp.dot(x, Wg_ref[...], preferred_element_type=jnp.float32)
        logits = logits + bg_ref[...]
        iota_e = jax.lax.broadcasted_iota(jnp.int32, logits.shape, 1)
        a1 = jnp.argmax(logits, axis=-1)[:, None]
        m1 = jnp.max(logits, axis=-1, keepdims=True)
        one1 = (iota_e == a1)
        l2 = jnp.where(one1, -jnp.inf, logits)
        a2 = jnp.argmax(l2, axis=-1)[:, None]
        m2 = jnp.max(l2, axis=-1, keepdims=True)
        w1 = 1.0 / (1.0 + jnp.exp(m2 - m1))
        w2 = 1.0 - w1
        one2 = (iota_e == a2)
        wfull_ref[...] = jnp.where(one1, w1, 0.0) + jnp.where(one2, w2, 0.0)

    x = x_ref[...]
    h = jnp.dot(x, W1_ref[0], preferred_element_type=jnp.float32) + b1_ref[0]
    h = h * (1.0 / (1.0 + jnp.exp(-h)))
    y = jnp.dot(h, W2_ref[0], preferred_element_type=jnp.float32) + b2_ref[0]

    wfull = wfull_ref[...]
    iota_e = jax.lax.broadcasted_iota(jnp.int32, wfull.shape, 1)
    w_e = jnp.sum(jnp.where(iota_e == e, wfull, 0.0), axis=1, keepdims=True)
    contrib = w_e * y

    @pl.when(e == 0)
    def _init():
        out_ref[...] = contrib

    @pl.when(e != 0)
    def _acc():
        out_ref[...] += contrib


def kernel(inputs, Wg, bg, W1, b1, W2, b2):
    B, S, D = inputs.shape
    E = Wg.shape[1]
    D_FF = W1.shape[2]
    T = B * S
    x = inputs.reshape(T, D)

    BLK = 512
    n_blk = T // BLK

    out = pl.pallas_call(
        functools.partial(_moe_body, n_exp=E),
        grid=(n_blk, E),
        in_specs=[
            pl.BlockSpec((BLK, D), lambda i, e: (i, 0)),
            pl.BlockSpec((D, E), lambda i, e: (0, 0)),
            pl.BlockSpec((1, E), lambda i, e: (0, 0)),
            pl.BlockSpec((1, D, D_FF), lambda i, e: (e, 0, 0)),
            pl.BlockSpec((1, 1, D_FF), lambda i, e: (e, 0, 0)),
            pl.BlockSpec((1, D_FF, D), lambda i, e: (e, 0, 0)),
            pl.BlockSpec((1, 1, D), lambda i, e: (e, 0, 0)),
        ],
        out_specs=pl.BlockSpec((BLK, D), lambda i, e: (i, 0)),
        out_shape=jax.ShapeDtypeStruct((T, D), jnp.float32),
        scratch_shapes=[pltpu.VMEM((BLK, E), jnp.float32)],
        compiler_params=pltpu.CompilerParams(
            dimension_semantics=("parallel", "arbitrary"),
        ),
    )(x, Wg, bg.reshape(1, E), W1, b1.reshape(E, 1, D_FF), W2, b2.reshape(E, 1, D))

    return out.reshape(B, S, D)
